# baseline (speedup 1.0000x reference)
import functools

import jax
import jax.numpy as jnp
from jax.experimental import pallas as pl
from jax.experimental.pallas import tpu as pltpu

_BF = jnp.bfloat16
_F32 = jnp.float32


def _bilinear_matrix(n_in, n_out):
    i = jnp.arange(n_out, dtype=_F32)
    src = i * (n_in - 1) / (n_out - 1)
    i0f = jnp.floor(src)
    frac = (src - i0f)[:, None]
    i0 = i0f.astype(jnp.int32)[:, None]
    i1 = jnp.minimum(i0 + 1, n_in - 1)
    j = jnp.arange(n_in, dtype=jnp.int32)[None, :]
    return (jnp.where(j == i0, 1.0 - frac, 0.0)
            + jnp.where(j == i1, frac, 0.0))


def _pick_batch_tile(n, h):
    nb = 1
    for cand in range(1, n + 1):
        if n % cand == 0 and cand * h <= 512:
            nb = cand
    return nb


def _decoder_kernel(nb, wco, xpad, xf_ref, sf_ref, uhb_ref, wgcat_ref,
                    bgs_ref, wo_ref, bo_ref, w1_ref, b1_ref, w2_ref, b2_ref,
                    o_ref):
    def mm(a, b):
        return jax.lax.dot_general(a, b, (((1,), (0,)), ((), ())),
                                   preferred_element_type=_F32)

    def shift_dn(q):
        z = jnp.zeros((nb, q.shape[1]), q.dtype)
        return jnp.concatenate([z, q[:-nb]], axis=0)

    def shift_up(q):
        z = jnp.zeros((nb, q.shape[1]), q.dtype)
        return jnp.concatenate([q[nb:], z], axis=0)

    h0, _, w0c = xf_ref.shape
    h = sf_ref.shape[0]
    xf = xf_ref[...].reshape(h0 * nb, w0c)
    xf = jnp.concatenate(
        [xf, jnp.zeros((h0 * nb, xpad - w0c), xf.dtype)], axis=1)
    sf = sf_ref[...].reshape(h * nb, wco)

    xh_b = mm(uhb_ref[...], xf).astype(_BF)

    cat1 = jnp.concatenate([xh_b, sf], axis=1)
    r = jnp.maximum(mm(cat1, wgcat_ref[...]) + bgs_ref[...], 0.0)
    z = mm(r.astype(_BF), wo_ref[...]) + bo_ref[...]
    att = 1.0 / (1.0 + jnp.exp(-z))

    cat2 = jnp.concatenate([xh_b, att.astype(_BF)], axis=1)
    p1 = mm(cat2, w1_ref[...])
    acc1 = (b1_ref[...] + p1[:, wco:2 * wco]
            + shift_dn(p1[:, :wco]) + shift_up(p1[:, 2 * wco:]))
    y1 = jnp.maximum(acc1, 0.0).astype(_BF)

    p2 = mm(y1, w2_ref[...])
    acc2 = (b2_ref[...] + p2[:, wco:2 * wco]
            + shift_dn(p2[:, :wco]) + shift_up(p2[:, 2 * wco:]))
    o_ref[...] = jnp.maximum(acc2, 0.0).astype(_BF).reshape(o_ref.shape)


def _forward(x_nchw, s_nchw, params):
    N, Cin, H0, W0 = x_nchw.shape
    _, Cout, H, W = s_nchw.shape

    W0C = W0 * Cin
    WCo = W * Cout
    XP = ((W0C + 127) // 128) * 128
    NB = _pick_batch_tile(N, H)
    G = N // NB
    M = NB * H
    MK = NB * H0

    xf = jnp.transpose(x_nchw.astype(_BF), (2, 0, 1, 3)).reshape(H0, N, W0C)
    sf = jnp.transpose(s_nchw.astype(_BF), (2, 0, 1, 3)).reshape(H, N, WCo)

    bf = lambda a: a.astype(_BF)
    Uh = _bilinear_matrix(H0, H)
    UwT = _bilinear_matrix(W0, W).T
    UhB = jnp.kron(Uh, jnp.eye(NB, dtype=_F32))
    bands = jnp.stack([jnp.eye(W, W, k=1 - dw, dtype=_F32)
                       for dw in range(3)])
    UwB = jnp.einsum("vb,dbw->dvw", UwT, bands)

    zxp = ((0, XP - W0C), (0, 0))
    WgU = jnp.einsum("vw,io->ivow", UwT, params["wg"]).reshape(W0C, WCo)
    Wgcat = jnp.concatenate(
        [jnp.pad(WgU, zxp), jnp.kron(params["ws"], jnp.eye(W, dtype=_F32))],
        axis=0)
    BDWo = jnp.kron(params["wo"], jnp.eye(W, dtype=_F32))
    bgs = jnp.repeat(params["bg"] + params["bs"], W)[None, :]
    bo = jnp.repeat(params["bo"], W)[None, :]

    w1 = params["w1"]
    W1x = jnp.einsum("dvw,hdio->hivow", UwB,
                     w1[:, :, :Cin, :]).reshape(3, W0C, WCo)
    W1x = jnp.pad(jnp.concatenate(list(W1x), axis=1), zxp)
    W1a = jnp.einsum("dvw,hdio->hivow", bands,
                     w1[:, :, Cin:, :]).reshape(3, WCo, WCo)
    W1a = jnp.concatenate(list(W1a), axis=1)
    W1cat = jnp.concatenate([W1x, W1a], axis=0)
    W2 = jnp.einsum("dvw,hdio->hivow", bands,
                    params["w2"]).reshape(3, WCo, WCo)
    W2 = jnp.concatenate(list(W2), axis=1)
    b1 = jnp.repeat(params["b1"], W)[None, :]
    b2 = jnp.repeat(params["b2"], W)[None, :]

    body = functools.partial(_decoder_kernel, NB, WCo, XP)

    out3 = pl.pallas_call(
        body,
        out_shape=jax.ShapeDtypeStruct((H, N, WCo), _BF),
        grid=(G,),
        in_specs=[
            pl.BlockSpec((H0, NB, W0C), lambda n: (0, n, 0)),
            pl.BlockSpec((H, NB, WCo), lambda n: (0, n, 0)),
            pl.BlockSpec((M, MK), lambda n: (0, 0)),
            pl.BlockSpec((XP + WCo, WCo), lambda n: (0, 0)),
            pl.BlockSpec((1, WCo), lambda n: (0, 0)),
            pl.BlockSpec((WCo, WCo), lambda n: (0, 0)),
            pl.BlockSpec((1, WCo), lambda n: (0, 0)),
            pl.BlockSpec((XP + WCo, 3 * WCo), lambda n: (0, 0)),
            pl.BlockSpec((1, WCo), lambda n: (0, 0)),
            pl.BlockSpec((WCo, 3 * WCo), lambda n: (0, 0)),
            pl.BlockSpec((1, WCo), lambda n: (0, 0)),
        ],
        out_specs=pl.BlockSpec((H, NB, WCo), lambda n: (0, n, 0)),
        compiler_params=pltpu.CompilerParams(
            dimension_semantics=("parallel",),
            vmem_limit_bytes=64 * 1024 * 1024),
    )(xf, sf, bf(UhB), bf(Wgcat), bgs, bf(BDWo), bo,
      bf(W1cat), b1, bf(W2), b2)

    return jnp.transpose(out3.reshape(H, N, Cout, W),
                         (1, 2, 0, 3)).astype(_F32)


_forward_jit = jax.jit(_forward)


def kernel(x, s, wg, bg, ws, bs, wo, bo, w1, b1, w2, b2):
    params = {
        "wg": wg, "bg": bg, "ws": ws, "bs": bs, "wo": wo, "bo": bo,
        "w1": w1, "b1": b1, "w2": w2, "b2": b2,
    }
    return _forward_jit(x, s, params)


# speedup vs baseline: 1.1070x; 1.1070x over previous
import functools

import jax
import jax.numpy as jnp
from jax.experimental import pallas as pl
from jax.experimental.pallas import tpu as pltpu

_BF = jnp.bfloat16
_F32 = jnp.float32


def _bilinear_matrix(n_in, n_out):
    i = jnp.arange(n_out, dtype=_F32)
    src = i * (n_in - 1) / (n_out - 1)
    i0f = jnp.floor(src)
    frac = (src - i0f)[:, None]
    i0 = i0f.astype(jnp.int32)[:, None]
    i1 = jnp.minimum(i0 + 1, n_in - 1)
    j = jnp.arange(n_in, dtype=jnp.int32)[None, :]
    return (jnp.where(j == i0, 1.0 - frac, 0.0)
            + jnp.where(j == i1, frac, 0.0))


def _pick_batch_tile(n, h):
    nb = 1
    for cand in range(1, n + 1):
        if n % cand == 0 and cand * h <= 1024:
            nb = cand
    return nb


def _decoder_kernel(nb, wco, xpad, xf_ref, sf_ref, uhb_ref, wgcat_ref,
                    bgs_ref, wo_ref, bo_ref, w1_ref, b1_ref, w2_ref, b2_ref,
                    o_ref):
    def mm(a, b):
        return jax.lax.dot_general(a, b, (((1,), (0,)), ((), ())),
                                   preferred_element_type=_F32)

    def shift_dn(q):
        z = jnp.zeros((nb, q.shape[1]), q.dtype)
        return jnp.concatenate([z, q[:-nb]], axis=0)

    def shift_up(q):
        z = jnp.zeros((nb, q.shape[1]), q.dtype)
        return jnp.concatenate([q[nb:], z], axis=0)

    h0, _, w0c = xf_ref.shape
    h = sf_ref.shape[0]
    xf = xf_ref[...].reshape(h0 * nb, w0c)
    xf = jnp.concatenate(
        [xf, jnp.zeros((h0 * nb, xpad - w0c), xf.dtype)], axis=1)
    sf = sf_ref[...].reshape(h * nb, wco)

    xh_b = mm(uhb_ref[...], xf).astype(_BF)

    cat1 = jnp.concatenate([xh_b, sf], axis=1)
    r = jnp.maximum(mm(cat1, wgcat_ref[...]) + bgs_ref[...], 0.0)
    z = mm(r.astype(_BF), wo_ref[...]) + bo_ref[...]
    att = 1.0 / (1.0 + jnp.exp(-z))

    cat2 = jnp.concatenate([xh_b, att.astype(_BF)], axis=1)
    catf = jnp.concatenate([shift_dn(cat2), cat2, shift_up(cat2)], axis=1)
    acc1 = mm(catf, w1_ref[...]) + b1_ref[...]
    y1 = jnp.maximum(acc1, 0.0).astype(_BF)

    caty = jnp.concatenate([shift_dn(y1), y1, shift_up(y1)], axis=1)
    acc2 = mm(caty, w2_ref[...]) + b2_ref[...]
    o_ref[...] = jnp.maximum(acc2, 0.0).astype(_BF).reshape(o_ref.shape)


def _forward(x_nchw, s_nchw, params):
    N, Cin, H0, W0 = x_nchw.shape
    _, Cout, H, W = s_nchw.shape

    W0C = W0 * Cin
    WCo = W * Cout
    XP = ((W0C + 127) // 128) * 128
    NB = _pick_batch_tile(N, H)
    G = N // NB
    M = NB * H
    MK = NB * H0

    xf = jnp.transpose(x_nchw.astype(_BF), (2, 0, 1, 3)).reshape(H0, N, W0C)
    sf = jnp.transpose(s_nchw.astype(_BF), (2, 0, 1, 3)).reshape(H, N, WCo)

    bf = lambda a: a.astype(_BF)
    Uh = _bilinear_matrix(H0, H)
    UwT = _bilinear_matrix(W0, W).T
    UhB = jnp.kron(Uh, jnp.eye(NB, dtype=_F32))
    bands = jnp.stack([jnp.eye(W, W, k=1 - dw, dtype=_F32)
                       for dw in range(3)])
    UwB = jnp.einsum("vb,dbw->dvw", UwT, bands)

    zxp = ((0, XP - W0C), (0, 0))
    WgU = jnp.einsum("vw,io->ivow", UwT, params["wg"]).reshape(W0C, WCo)
    Wgcat = jnp.concatenate(
        [jnp.pad(WgU, zxp), jnp.kron(params["ws"], jnp.eye(W, dtype=_F32))],
        axis=0)
    BDWo = jnp.kron(params["wo"], jnp.eye(W, dtype=_F32))
    bgs = jnp.repeat(params["bg"] + params["bs"], W)[None, :]
    bo = jnp.repeat(params["bo"], W)[None, :]

    w1 = params["w1"]
    W1x = jnp.pad(jnp.einsum("dvw,hdio->hivow", UwB,
                             w1[:, :, :Cin, :]).reshape(3, W0C, WCo),
                  ((0, 0),) + zxp)
    W1a = jnp.einsum("dvw,hdio->hivow", bands,
                     w1[:, :, Cin:, :]).reshape(3, WCo, WCo)
    W1cat = jnp.concatenate([W1x, W1a], axis=1)
    W1cat = W1cat.reshape(3 * (XP + WCo), WCo)
    W2 = jnp.einsum("dvw,hdio->hivow", bands,
                    params["w2"]).reshape(3 * WCo, WCo)
    b1 = jnp.repeat(params["b1"], W)[None, :]
    b2 = jnp.repeat(params["b2"], W)[None, :]

    body = functools.partial(_decoder_kernel, NB, WCo, XP)

    out3 = pl.pallas_call(
        body,
        out_shape=jax.ShapeDtypeStruct((H, N, WCo), _BF),
        grid=(G,),
        in_specs=[
            pl.BlockSpec((H0, NB, W0C), lambda n: (0, n, 0)),
            pl.BlockSpec((H, NB, WCo), lambda n: (0, n, 0)),
            pl.BlockSpec((M, MK), lambda n: (0, 0)),
            pl.BlockSpec((XP + WCo, WCo), lambda n: (0, 0)),
            pl.BlockSpec((1, WCo), lambda n: (0, 0)),
            pl.BlockSpec((WCo, WCo), lambda n: (0, 0)),
            pl.BlockSpec((1, WCo), lambda n: (0, 0)),
            pl.BlockSpec((3 * (XP + WCo), WCo), lambda n: (0, 0)),
            pl.BlockSpec((1, WCo), lambda n: (0, 0)),
            pl.BlockSpec((3 * WCo, WCo), lambda n: (0, 0)),
            pl.BlockSpec((1, WCo), lambda n: (0, 0)),
        ],
        out_specs=pl.BlockSpec((H, NB, WCo), lambda n: (0, n, 0)),
        compiler_params=pltpu.CompilerParams(
            dimension_semantics=("parallel",),
            vmem_limit_bytes=64 * 1024 * 1024),
    )(xf, sf, bf(UhB), bf(Wgcat), bgs, bf(BDWo), bo,
      bf(W1cat), b1, bf(W2), b2)

    return jnp.transpose(out3.reshape(H, N, Cout, W),
                         (1, 2, 0, 3)).astype(_F32)


_forward_jit = jax.jit(_forward)


def kernel(x, s, wg, bg, ws, bs, wo, bo, w1, b1, w2, b2):
    params = {
        "wg": wg, "bg": bg, "ws": ws, "bs": bs, "wo": wo, "bo": bo,
        "w1": w1, "b1": b1, "w2": w2, "b2": b2,
    }
    return _forward_jit(x, s, params)


# speedup vs baseline: 1.1223x; 1.0138x over previous
import functools

import jax
import jax.numpy as jnp
from jax.experimental import pallas as pl
from jax.experimental.pallas import tpu as pltpu

_BF = jnp.bfloat16
_F32 = jnp.float32


def _bilinear_matrix(n_in, n_out):
    i = jnp.arange(n_out, dtype=_F32)
    src = i * (n_in - 1) / (n_out - 1)
    i0f = jnp.floor(src)
    frac = (src - i0f)[:, None]
    i0 = i0f.astype(jnp.int32)[:, None]
    i1 = jnp.minimum(i0 + 1, n_in - 1)
    j = jnp.arange(n_in, dtype=jnp.int32)[None, :]
    return (jnp.where(j == i0, 1.0 - frac, 0.0)
            + jnp.where(j == i1, frac, 0.0))


def _pick_batch_tile(n, h):
    nb = 1
    for cand in range(1, n + 1):
        if n % cand == 0 and cand * h <= 1024:
            nb = cand
    return nb


def _decoder_kernel(nb, wco, xpad, xf_ref, sf_ref, uhb_ref, wgcat_ref,
                    wo_ref, w1_ref, w2_ref, bias_ref, o_ref):
    def mm(a, b):
        return jax.lax.dot_general(a, b, (((1,), (0,)), ((), ())),
                                   preferred_element_type=_F32)

    def shift_dn(q):
        z = jnp.zeros((nb, q.shape[1]), q.dtype)
        return jnp.concatenate([z, q[:-nb]], axis=0)

    def shift_up(q):
        z = jnp.zeros((nb, q.shape[1]), q.dtype)
        return jnp.concatenate([q[nb:], z], axis=0)

    h0, _, w0c = xf_ref.shape
    h = sf_ref.shape[0]
    xf = xf_ref[...].reshape(h0 * nb, w0c)
    xf = jnp.concatenate(
        [xf, jnp.zeros((h0 * nb, xpad - w0c), xf.dtype)], axis=1)
    sf = sf_ref[...].reshape(h * nb, wco)

    b = bias_ref[...]

    xh_b = mm(uhb_ref[...], xf).astype(_BF)

    cat1 = jnp.concatenate([xh_b, sf], axis=1)
    r = jnp.maximum(mm(cat1, wgcat_ref[...]) + b[0:1], 0.0)
    z = mm(r.astype(_BF), wo_ref[...]) + b[1:2]
    att = 1.0 / (1.0 + jnp.exp(-z))

    cat2 = jnp.concatenate([xh_b, att.astype(_BF)], axis=1)
    catf = jnp.concatenate([shift_dn(cat2), cat2, shift_up(cat2)], axis=1)
    acc1 = mm(catf, w1_ref[...]) + b[2:3]
    y1 = jnp.maximum(acc1, 0.0).astype(_BF)

    caty = jnp.concatenate([shift_dn(y1), y1, shift_up(y1)], axis=1)
    acc2 = mm(caty, w2_ref[...]) + b[3:4]
    o_ref[...] = jnp.maximum(acc2, 0.0).astype(_BF).reshape(o_ref.shape)


def _forward(x_nchw, s_nchw, params):
    N, Cin, H0, W0 = x_nchw.shape
    _, Cout, H, W = s_nchw.shape

    W0C = W0 * Cin
    WCo = W * Cout
    XP = ((W0C + 127) // 128) * 128
    NB = _pick_batch_tile(N, H)
    G = N // NB
    M = NB * H
    MK = NB * H0

    xf = jnp.transpose(x_nchw, (2, 0, 1, 3)).reshape(H0, N, W0C).astype(_BF)
    sf = jnp.transpose(s_nchw.astype(_BF), (2, 0, 1, 3)).reshape(H, N, WCo)

    bf = lambda a: a.astype(_BF)
    Uh = _bilinear_matrix(H0, H)
    UwT = _bilinear_matrix(W0, W).T
    UhB = jnp.kron(Uh, jnp.eye(NB, dtype=_F32))
    bands = jnp.stack([jnp.eye(W, W, k=1 - dw, dtype=_F32)
                       for dw in range(3)])
    UwB = jnp.einsum("vb,dbw->dvw", UwT, bands)

    zxp = ((0, XP - W0C), (0, 0))
    WgU = jnp.einsum("vw,io->ivow", UwT, params["wg"]).reshape(W0C, WCo)
    Wgcat = jnp.concatenate(
        [jnp.pad(WgU, zxp), jnp.kron(params["ws"], jnp.eye(W, dtype=_F32))],
        axis=0)
    BDWo = jnp.kron(params["wo"], jnp.eye(W, dtype=_F32))
    biases = jnp.repeat(
        jnp.stack([params["bg"] + params["bs"], params["bo"],
                   params["b1"], params["b2"]]), W, axis=1)

    w1 = params["w1"]
    W1x = jnp.pad(jnp.einsum("dvw,hdio->hivow", UwB,
                             w1[:, :, :Cin, :]).reshape(3, W0C, WCo),
                  ((0, 0),) + zxp)
    W1a = jnp.einsum("dvw,hdio->hivow", bands,
                     w1[:, :, Cin:, :]).reshape(3, WCo, WCo)
    W1cat = jnp.concatenate([W1x, W1a], axis=1)
    W1cat = W1cat.reshape(3 * (XP + WCo), WCo)
    W2 = jnp.einsum("dvw,hdio->hivow", bands,
                    params["w2"]).reshape(3 * WCo, WCo)

    body = functools.partial(_decoder_kernel, NB, WCo, XP)

    out3 = pl.pallas_call(
        body,
        out_shape=jax.ShapeDtypeStruct((H, N, WCo), _BF),
        grid=(G,),
        in_specs=[
            pl.BlockSpec((H0, NB, W0C), lambda n: (0, n, 0)),
            pl.BlockSpec((H, NB, WCo), lambda n: (0, n, 0)),
            pl.BlockSpec((M, MK), lambda n: (0, 0)),
            pl.BlockSpec((XP + WCo, WCo), lambda n: (0, 0)),
            pl.BlockSpec((WCo, WCo), lambda n: (0, 0)),
            pl.BlockSpec((3 * (XP + WCo), WCo), lambda n: (0, 0)),
            pl.BlockSpec((3 * WCo, WCo), lambda n: (0, 0)),
            pl.BlockSpec((4, WCo), lambda n: (0, 0)),
        ],
        out_specs=pl.BlockSpec((H, NB, WCo), lambda n: (0, n, 0)),
        compiler_params=pltpu.CompilerParams(
            dimension_semantics=("parallel",),
            vmem_limit_bytes=64 * 1024 * 1024),
    )(xf, sf, bf(UhB), bf(Wgcat), bf(BDWo), bf(W1cat), bf(W2), biases)

    return jnp.transpose(out3.reshape(H, N, Cout, W),
                         (1, 2, 0, 3)).astype(_F32)


_forward_jit = jax.jit(_forward)


def kernel(x, s, wg, bg, ws, bs, wo, bo, w1, b1, w2, b2):
    params = {
        "wg": wg, "bg": bg, "ws": ws, "bs": bs, "wo": wo, "bo": bo,
        "w1": w1, "b1": b1, "w2": w2, "b2": b2,
    }
    return _forward_jit(x, s, params)
